# baseline (speedup 1.0000x reference)
import functools
import math

import jax
import jax.numpy as jnp
from jax import lax
from jax.experimental import pallas as pl
from jax.experimental.pallas import tpu as pltpu
from jax.experimental.pallas import tpu_sc as plsc

DM = 64
SCALE = math.sqrt(DM)

NC = 2
NS = 16
NW = NC * NS
L = 16

IPG = 128
GPC = 4
CHUNK = IPG * GPC


def _emb_lookup(table, idx2d):
    n_rows, _ = idx2d.shape
    B = n_rows * IPG
    rows_per_w = n_rows // NW
    n_chunks = rows_per_w // GPC
    assert rows_per_w % GPC == 0

    mesh = plsc.VectorSubcoreMesh(core_axis_name="c", subcore_axis_name="s")

    @functools.partial(
        pl.kernel,
        mesh=mesh,
        out_type=jax.ShapeDtypeStruct((B, DM), jnp.float32),
        scratch_types=[
            pltpu.VMEM((GPC, IPG), jnp.int32),
            pltpu.VMEM((CHUNK, DM), jnp.float32),
            pltpu.SemaphoreType.DMA,
        ],
        compiler_params=pltpu.CompilerParams(use_tc_tiling_on_sc=False),
    )
    def k(table_hbm, idx_hbm, out_hbm, idx_v, rows_v, gsem):
        wid = lax.axis_index("s") * NC + lax.axis_index("c")
        base_row = wid * rows_per_w

        def chunk_body(g, carry):
            row_off = base_row + g * GPC
            pltpu.sync_copy(idx_hbm.at[pl.ds(row_off, GPC)], idx_v)
            descs = []
            for j in range(GPC):
                descs.append(
                    pltpu.async_copy(
                        table_hbm.at[idx_v.at[j]],
                        rows_v.at[pl.ds(j * IPG, IPG)],
                        gsem,
                    )
                )
            for d in descs:
                d.wait()

            def scale_row(i, c):
                for j in range(DM // L):
                    sl = pl.ds(j * L, L)
                    rows_v[i, sl] = rows_v[i, sl] * SCALE
                return c

            lax.fori_loop(0, CHUNK, scale_row, 0, unroll=2)
            pltpu.sync_copy(rows_v, out_hbm.at[pl.ds(row_off * IPG, CHUNK)])
            return carry

        lax.fori_loop(0, n_chunks, chunk_body, 0)

    return k(table, idx2d)


def kernel(x, table):
    b, s = x.shape
    idx2d = x.reshape(-1, IPG)
    out = _emb_lookup(table, idx2d)
    return out.reshape(b, s, DM)


# speedup vs baseline: 1.0753x; 1.0753x over previous
import functools
import math

import jax
import jax.numpy as jnp
from jax import lax
from jax.experimental import pallas as pl
from jax.experimental.pallas import tpu as pltpu
from jax.experimental.pallas import tpu_sc as plsc

DM = 64
SCALE = math.sqrt(DM)

NC = 2
NS = 16
NW = NC * NS
L = 16

IPG = 128
GPC = 4
CHUNK = IPG * GPC


def _emb_lookup(table, idx2d):
    n_rows, _ = idx2d.shape
    B = n_rows * IPG
    rows_per_w = n_rows // NW
    n_chunks = rows_per_w // GPC
    assert rows_per_w % GPC == 0 and n_chunks % 2 == 0

    mesh = plsc.VectorSubcoreMesh(core_axis_name="c", subcore_axis_name="s")

    @functools.partial(
        pl.kernel,
        mesh=mesh,
        out_type=jax.ShapeDtypeStruct((B, DM), jnp.float32),
        scratch_types=[
            pltpu.VMEM((2, GPC, IPG), jnp.int32),
            pltpu.VMEM((2, CHUNK, DM), jnp.float32),
            pltpu.SemaphoreType.DMA,
            pltpu.SemaphoreType.DMA,
            pltpu.SemaphoreType.DMA,
            pltpu.SemaphoreType.DMA,
        ],
        compiler_params=pltpu.CompilerParams(use_tc_tiling_on_sc=False),
    )
    def k(table_hbm, idx_hbm, out_hbm, idx_v, rows_v, gsem0, gsem1, wsem0,
          wsem1):
        gsems = (gsem0, gsem1)
        wsems = (wsem0, wsem1)
        wid = lax.axis_index("s") * NC + lax.axis_index("c")
        base_row = wid * rows_per_w

        def fire(c, bb):
            pltpu.sync_copy(idx_hbm.at[pl.ds(base_row + c * GPC, GPC)],
                            idx_v.at[bb])
            for j in range(GPC):
                pltpu.async_copy(
                    table_hbm.at[idx_v.at[bb, j]],
                    rows_v.at[bb, pl.ds(j * IPG, IPG)],
                    gsems[bb],
                )

        def drain(sem, bb):
            pltpu.make_async_copy(out_hbm.at[pl.ds(0, CHUNK)],
                                  rows_v.at[bb], sem).wait()

        fire(0, 0)

        def pair(t, carry):
            go = t * 2
            for b in (0, 1):
                c = go + b
                nb = 1 - b

                @pl.when(c + 1 < n_chunks)
                def _():
                    @pl.when(c >= 1)
                    def _():
                        drain(wsems[nb], nb)
                    fire(c + 1, nb)

                drain(gsems[b], b)

                def scale_row(i, cr):
                    for j in range(DM // L):
                        sl = pl.ds(j * L, L)
                        rows_v[b, i, sl] = rows_v[b, i, sl] * SCALE
                    return cr

                lax.fori_loop(0, CHUNK, scale_row, 0, unroll=4)
                pltpu.async_copy(
                    rows_v.at[b],
                    out_hbm.at[pl.ds((base_row + c * GPC) * IPG, CHUNK)],
                    wsems[b],
                )
            return carry

        lax.fori_loop(0, n_chunks // 2, pair, 0)
        drain(wsems[0], 0)
        drain(wsems[1], 1)

    return k(table, idx2d)


def kernel(x, table):
    b, s = x.shape
    idx2d = x.reshape(-1, IPG)
    out = _emb_lookup(table, idx2d)
    return out.reshape(b, s, DM)


# speedup vs baseline: 1.0833x; 1.0075x over previous
import functools
import math

import jax
import jax.numpy as jnp
from jax import lax
from jax.experimental import pallas as pl
from jax.experimental.pallas import tpu as pltpu
from jax.experimental.pallas import tpu_sc as plsc

DM = 64
SCALE = math.sqrt(DM)

NC = 2
NS = 16
NW = NC * NS
L = 16

CB = 4


def _emb_lookup(table, x):
    nb, sl = x.shape
    rows_per_w = nb // NW
    n_chunks = rows_per_w // CB
    assert rows_per_w % CB == 0 and n_chunks % 2 == 0
    g0 = 128
    g1 = sl - g0

    mesh = plsc.VectorSubcoreMesh(core_axis_name="c", subcore_axis_name="s")

    @functools.partial(
        pl.kernel,
        mesh=mesh,
        out_type=jax.ShapeDtypeStruct((nb, sl, DM), jnp.float32),
        scratch_types=[
            pltpu.VMEM((2, CB, sl), jnp.int32),
            pltpu.VMEM((2, CB, sl, DM), jnp.float32),
            pltpu.SemaphoreType.DMA,
            pltpu.SemaphoreType.DMA,
            pltpu.SemaphoreType.DMA,
            pltpu.SemaphoreType.DMA,
        ],
        compiler_params=pltpu.CompilerParams(use_tc_tiling_on_sc=False),
    )
    def k(table_hbm, x_hbm, out_hbm, idx_v, rows_v, gsem0, gsem1, wsem0,
          wsem1):
        gsems = (gsem0, gsem1)
        wsems = (wsem0, wsem1)
        wid = lax.axis_index("s") * NC + lax.axis_index("c")
        base = wid * rows_per_w

        def fire(c, bb):
            b0 = base + c * CB
            pltpu.sync_copy(x_hbm.at[pl.ds(b0, CB)], idx_v.at[bb])
            for r in range(CB):
                pltpu.async_copy(
                    table_hbm.at[idx_v.at[bb, r, pl.ds(0, g0)]],
                    rows_v.at[bb, r, pl.ds(0, g0)],
                    gsems[bb],
                )
                pltpu.async_copy(
                    table_hbm.at[idx_v.at[bb, r, pl.ds(g0, g1)]],
                    rows_v.at[bb, r, pl.ds(g0, g1)],
                    gsems[bb],
                )

        def drain(sem, bb):
            pltpu.make_async_copy(out_hbm.at[pl.ds(0, CB)],
                                  rows_v.at[bb], sem).wait()

        fire(0, 0)

        def pair(t, carry):
            go = t * 2
            for b in (0, 1):
                c = go + b
                nb_ = 1 - b

                @pl.when(c + 1 < n_chunks)
                def _():
                    @pl.when(c >= 1)
                    def _():
                        drain(wsems[nb_], nb_)
                    fire(c + 1, nb_)

                drain(gsems[b], b)

                for r in range(CB):

                    def scale_row(i, cr, _b=b, _r=r):
                        for j in range(DM // L):
                            s = pl.ds(j * L, L)
                            rows_v[_b, _r, i, s] = rows_v[_b, _r, i, s] * SCALE
                        return cr

                    lax.fori_loop(0, sl, scale_row, 0, unroll=4)

                pltpu.async_copy(
                    rows_v.at[b],
                    out_hbm.at[pl.ds(base + c * CB, CB)],
                    wsems[b],
                )
            return carry

        lax.fori_loop(0, n_chunks // 2, pair, 0)
        drain(wsems[0], 0)
        drain(wsems[1], 1)

    return k(table, x)


def kernel(x, table):
    return _emb_lookup(table, x)
